# baseline (speedup 1.0000x reference)
import functools

import jax
import jax.numpy as jnp
from jax import lax
from jax.experimental import pallas as pl
from jax.experimental.pallas import tpu as pltpu
from jax.experimental.pallas import tpu_sc as plsc


def _linear_body(s_ref, x_ref, w_ref, b_ref, o_ref):
    o_ref[...] = (
        jnp.dot(x_ref[...], w_ref[...], preferred_element_type=jnp.float32,
                  precision=lax.Precision.HIGHEST)
        + b_ref[...]
    ) * s_ref[0]


def _linear(x, w, b, sc, rb):
    n, din = x.shape
    dout = w.shape[1]
    return pl.pallas_call(
        _linear_body,
        grid=(n // rb,),
        in_specs=[
            pl.BlockSpec(memory_space=pltpu.SMEM),
            pl.BlockSpec((rb, din), lambda i: (i, 0)),
            pl.BlockSpec((din, dout), lambda i: (0, 0)),
            pl.BlockSpec((1, dout), lambda i: (0, 0)),
        ],
        out_specs=pl.BlockSpec((rb, dout), lambda i: (i, 0)),
        out_shape=jax.ShapeDtypeStruct((n, dout), jnp.float32),
    )(sc.reshape(1), x, w, b.reshape(1, -1))


def _edge_embed_body(s_ref, ea_ref, wl_ref, bl_ref, wh_ref, bh_ref, o_ref):
    sc = s_ref[0]
    lo = (jnp.dot(ea_ref[...], wl_ref[...],
                  preferred_element_type=jnp.float32,
                  precision=lax.Precision.HIGHEST) + bl_ref[...])
    hi = (jnp.dot(ea_ref[...], wh_ref[...],
                  preferred_element_type=jnp.float32,
                  precision=lax.Precision.HIGHEST) + bh_ref[...])
    ql = jnp.clip(lo * sc, -32767.0, 32767.0).astype(jnp.int32)
    qh = jnp.clip(hi * sc, -32767.0, 32767.0).astype(jnp.int32)
    o_ref[...] = (qh << 16) | (ql & jnp.int32(0xFFFF))


def _edge_embed(ea, w, b, sc, e_pad, eb):
    de, h = w.shape
    cols = jnp.arange(h // 2)
    g, kk = cols // 16, cols % 16
    perm_lo = g * 32 + kk
    perm_hi = g * 32 + 16 + kk
    return pl.pallas_call(
        _edge_embed_body,
        grid=(e_pad // eb,),
        in_specs=[
            pl.BlockSpec(memory_space=pltpu.SMEM),
            pl.BlockSpec((eb, de), lambda i: (i, 0)),
            pl.BlockSpec((de, h // 2), lambda i: (0, 0)),
            pl.BlockSpec((1, h // 2), lambda i: (0, 0)),
            pl.BlockSpec((de, h // 2), lambda i: (0, 0)),
            pl.BlockSpec((1, h // 2), lambda i: (0, 0)),
        ],
        out_specs=pl.BlockSpec((eb, h // 2), lambda i: (i, 0)),
        out_shape=jax.ShapeDtypeStruct((e_pad, h // 2), jnp.int32),
    )(sc.reshape(1), ea, w[:, perm_lo], b[perm_lo].reshape(1, -1),
      w[:, perm_hi], b[perm_hi].reshape(1, -1))


def _absmax_body(x_ref, o_ref):
    m = jnp.max(jnp.abs(x_ref[...]))

    @pl.when(pl.program_id(0) == 0)
    def _():
        o_ref[0, 0] = 0.0
    o_ref[0, 0] = jnp.maximum(o_ref[0, 0], m)


def _absmax(x, rb):
    n, d = x.shape
    return pl.pallas_call(
        _absmax_body,
        grid=(n // rb,),
        in_specs=[pl.BlockSpec((rb, d), lambda i: (i, 0))],
        out_specs=pl.BlockSpec(memory_space=pltpu.SMEM),
        out_shape=jax.ShapeDtypeStruct((1, 1), jnp.float32),
    )(x)


def _gine_mlp_body(eps_ref, x_ref, agg_ref, w1_ref, b1_ref, w2_ref, b2_ref,
                   g_ref, bb_ref, o_ref):
    eps = eps_ref[0]
    inv_s = eps_ref[1]
    s_out = eps_ref[2]
    dd = x_ref.shape[1]
    agg = agg_ref[0][:, :dd] + agg_ref[1][:, :dd]
    h = ((1.0 + eps) * x_ref[...] + agg) * inv_s
    t = jnp.maximum(
        jnp.dot(h, w1_ref[...], preferred_element_type=jnp.float32,
                  precision=lax.Precision.HIGHEST)
        + b1_ref[...], 0.0)
    o = (jnp.dot(t, w2_ref[...], preferred_element_type=jnp.float32,
                  precision=lax.Precision.HIGHEST)
         + b2_ref[...])
    mu = jnp.mean(o, axis=-1, keepdims=True)
    oc = o - mu
    var = jnp.mean(oc * oc, axis=-1, keepdims=True)
    o = oc * lax.rsqrt(var + 1e-5) * g_ref[...] + bb_ref[...]
    o_ref[...] = jnp.maximum(o, 0.0) * s_out


def _gine_mlp(xx, agg2, eps, w1, b1, w2, b2, g, bb, rb):
    n, dd = xx.shape
    da = agg2.shape[2]
    d2 = w1.shape[1]
    return pl.pallas_call(
        _gine_mlp_body,
        grid=(n // rb,),
        in_specs=[
            pl.BlockSpec(memory_space=pltpu.SMEM),
            pl.BlockSpec((rb, dd), lambda i: (i, 0)),
            pl.BlockSpec((2, rb, da), lambda i: (0, i, 0)),
            pl.BlockSpec((dd, d2), lambda i: (0, 0)),
            pl.BlockSpec((1, d2), lambda i: (0, 0)),
            pl.BlockSpec((d2, dd), lambda i: (0, 0)),
            pl.BlockSpec((1, dd), lambda i: (0, 0)),
            pl.BlockSpec((1, dd), lambda i: (0, 0)),
            pl.BlockSpec((1, dd), lambda i: (0, 0)),
        ],
        out_specs=pl.BlockSpec((rb, dd), lambda i: (i, 0)),
        out_shape=jax.ShapeDtypeStruct((n, dd), jnp.float32),
    )(eps, xx, agg2, w1, b1.reshape(1, -1), w2,
      b2.reshape(1, -1), g.reshape(1, -1), bb.reshape(1, -1))


def _pool_body(s_ref, a2f_ref, h_ref, w1_ref, b1_ref, w2_ref, b2_ref,
               o_ref):
    fb, _ = o_ref.shape
    nfull = h_ref.shape[0]
    f0 = pl.program_id(0) * fb
    fid = lax.broadcasted_iota(jnp.int32, (fb, nfull), 0) + f0
    oh = (fid == a2f_ref[...]).astype(jnp.float32)
    sums = jnp.dot(oh, h_ref[...], preferred_element_type=jnp.float32,
                  precision=lax.Precision.HIGHEST)
    cnt = jnp.sum(oh, axis=1, keepdims=True)
    m = sums / jnp.maximum(cnt, 1.0)
    t = (jnp.dot(m, w1_ref[...], preferred_element_type=jnp.float32,
                  precision=lax.Precision.HIGHEST)
         + b1_ref[...])
    o_ref[...] = (jnp.dot(t, w2_ref[...], preferred_element_type=jnp.float32,
                  precision=lax.Precision.HIGHEST)
                  + b2_ref[...]) * s_ref[0]


def _pool(a2f, h_atom, w1, b1, w2, b2, sc, nf, fb):
    n, dd = h_atom.shape
    hf = w1.shape[1]
    return pl.pallas_call(
        _pool_body,
        grid=(nf // fb,),
        in_specs=[
            pl.BlockSpec(memory_space=pltpu.SMEM),
            pl.BlockSpec((1, n), lambda i: (0, 0)),
            pl.BlockSpec((n, dd), lambda i: (0, 0)),
            pl.BlockSpec((dd, hf), lambda i: (0, 0)),
            pl.BlockSpec((1, hf), lambda i: (0, 0)),
            pl.BlockSpec((hf, hf), lambda i: (0, 0)),
            pl.BlockSpec((1, hf), lambda i: (0, 0)),
        ],
        out_specs=pl.BlockSpec((fb, hf), lambda i: (i, 0)),
        out_shape=jax.ShapeDtypeStruct((nf, hf), jnp.float32),
    )(sc.reshape(1), a2f.reshape(1, -1), h_atom, w1, b1.reshape(1, -1), w2,
      b2.reshape(1, -1))


def _final_body(hf_ref, fb_ref, cond_ref, fow1_ref, fob1_ref, fow2_ref,
                fob2_ref, cpw1_ref, cpb1_ref, cpw2_ref, cpb2_ref, gmw1_ref,
                gmb1_ref, gmw2_ref, gmb2_ref, dg_ref, dl_ref):
    hf = hf_ref[...]
    nb = dg_ref.shape[0]
    nf = hf.shape[0]
    hm = hf_ref.shape[1]
    t = jnp.maximum(
        jnp.dot(hf, fow1_ref[...], preferred_element_type=jnp.float32,
                  precision=lax.Precision.HIGHEST)
        + fob1_ref[...], 0.0)
    deltas = (jnp.dot(t, fow2_ref[...], preferred_element_type=jnp.float32,
                  precision=lax.Precision.HIGHEST)
              + fob2_ref[...])
    dl_ref[...] = deltas
    oh = (lax.broadcasted_iota(jnp.int32, (nb, nf), 0)
          == fb_ref[...]).astype(jnp.float32)
    fs = jnp.dot(oh, deltas, preferred_element_type=jnp.float32,
                  precision=lax.Precision.HIGHEST)
    cnt = jnp.sum(oh, axis=1, keepdims=True)
    hmol = (jnp.dot(oh, hf, preferred_element_type=jnp.float32,
                  precision=lax.Precision.HIGHEST)
            / jnp.maximum(cnt, 1.0))
    hc = jnp.maximum(
        jnp.dot(cond_ref[...], cpw1_ref[...],
                preferred_element_type=jnp.float32,
                  precision=lax.Precision.HIGHEST) + cpb1_ref[...], 0.0)
    hc = (jnp.dot(hc, cpw2_ref[...], preferred_element_type=jnp.float32,
                  precision=lax.Precision.HIGHEST)
          + cpb2_ref[...])
    gw = gmw1_ref[...]
    z = (jnp.dot(hmol, gw[:hm, :], preferred_element_type=jnp.float32,
                  precision=lax.Precision.HIGHEST)
         + jnp.dot(hc, gw[hm:, :], preferred_element_type=jnp.float32,
                  precision=lax.Precision.HIGHEST)
         + gmb1_ref[...])
    g = (jnp.dot(jnp.maximum(z, 0.0), gmw2_ref[...],
                 preferred_element_type=jnp.float32,
                  precision=lax.Precision.HIGHEST) + gmb2_ref[...])
    dg_ref[...] = fs + g


def _final(h_frag, frag_batch, cond, p):
    nf, hm = h_frag.shape
    nb = cond.shape[0]
    return pl.pallas_call(
        _final_body,
        out_shape=[
            jax.ShapeDtypeStruct((nb, 1), jnp.float32),
            jax.ShapeDtypeStruct((nf, 1), jnp.float32),
        ],
    )(h_frag, frag_batch.reshape(1, -1), cond,
      p['fo_W1'], p['fo_b1'].reshape(1, -1),
      p['fo_W2'], p['fo_b2'].reshape(1, -1),
      p['cp_W1'], p['cp_b1'].reshape(1, -1),
      p['cp_W2'], p['cp_b2'].reshape(1, -1),
      p['gm_W1'], p['gm_b1'].reshape(1, -1),
      p['gm_W2'], p['gm_b2'].reshape(1, -1))


_ZR = 8


def _sc_edge_pass(x, ea, src3d, dst3d, e_pad, n_acc, ch, kslab):
    d = x.shape[1]
    epw = e_pad // 32
    chunks = epw // ch
    nslabs = chunks // kslab
    rpt = n_acc // 16
    zcopies = rpt // _ZR
    dpe = d // 16

    mesh = plsc.VectorSubcoreMesh(core_axis_name="c", subcore_axis_name="s")

    @functools.partial(
        pl.kernel,
        mesh=mesh,
        out_type=jax.ShapeDtypeStruct((2, n_acc, d), jnp.float32),
        scratch_types=[
            pltpu.VMEM((kslab, ch), jnp.int32),
            pltpu.VMEM((kslab, ch), jnp.int32),
            pltpu.VMEM((ch, d), jnp.float32),
            pltpu.VMEM((ch, d // 2), jnp.int32),
            pltpu.VMEM((ch, d), jnp.float32),
            pltpu.VMEM((ch, d // 2), jnp.int32),
            pltpu.VMEM((_ZR, d), jnp.float32),
            pltpu.VMEM_SHARED((n_acc, d), jnp.float32),
            pltpu.SemaphoreType.DMA,
            pltpu.SemaphoreType.DMA,
            pltpu.SemaphoreType.DMA,
            pltpu.SemaphoreType.DMA,
        ],
    )
    def k(x_hbm, ea_hbm, src_hbm, dst_hbm, out_hbm, sslab, dslab,
          rowsA, eabA, rowsB, eabB, zbuf, acc, semA, semB, ssemA, ssemB):
        c = lax.axis_index("c")
        s = lax.axis_index("s")

        def zrow(r, carry):
            for j in range(dpe):
                zbuf[r, pl.ds(j * 16, 16)] = jnp.zeros((16,), jnp.float32)
            return carry
        lax.fori_loop(0, _ZR, zrow, 0)

        def zcp(kk, carry):
            pltpu.sync_copy(zbuf, acc.at[pl.ds(s * rpt + kk * _ZR, _ZR)])
            return carry
        lax.fori_loop(0, zcopies, zcp, 0)
        plsc.subcore_barrier()

        chunk0 = c * (e_pad // (2 * ch)) + s * chunks
        bufs = [(rowsA, eabA, semA, ssemA), (rowsB, eabB, semB, ssemB)]

        nslabs_ = chunks // kslab

        def slab(si, carry):
            row0 = chunk0 + si * kslab
            slab_id = (c * 16 + s) * nslabs_ + si
            pltpu.sync_copy(src_hbm.at[slab_id], sslab)
            pltpu.sync_copy(dst_hbm.at[slab_id], dslab)

            def issue(j, rows, eab, sem):
                off = (row0 + j) * ch
                pltpu.async_copy(ea_hbm.at[pl.ds(off, ch)], eab, sem)
                pltpu.async_copy(x_hbm.at[sslab.at[j]], rows, sem)

            issue(0, rowsA, eabA, semA)
            for j in range(kslab):
                rows, eab, sem, ssem = bufs[j % 2]
                orows, _, _, ossem = bufs[1 - j % 2]
                off = (row0 + j) * ch
                pltpu.make_async_copy(ea_hbm.at[pl.ds(off, ch)], eab,
                                      sem).wait()
                pltpu.make_async_copy(x_hbm.at[sslab.at[j]], rows,
                                      sem).wait()

                def rrow(r, cc):
                    for jj in range(d // 32):
                        w = eab[r, pl.ds(jj * 16, 16)]
                        lo = ((w << 16) >> 16).astype(jnp.float32)
                        hi = (w >> 16).astype(jnp.float32)
                        sl0 = pl.ds(jj * 32, 16)
                        sl1 = pl.ds(jj * 32 + 16, 16)
                        rows[r, sl0] = jnp.maximum(rows[r, sl0] + lo, 0.0)
                        rows[r, sl1] = jnp.maximum(rows[r, sl1] + hi, 0.0)
                    return cc
                lax.fori_loop(0, ch, rrow, 0)

                if j >= 1:
                    pltpu.make_async_copy(orows, acc.at[dslab.at[j - 1]],
                                          ossem).wait()
                pltpu.async_copy(rows, acc.at[dslab.at[j]], ssem, add=True)
                if j + 1 < kslab:
                    issue(j + 1, *bufs[1 - j % 2][:3])

            lrows, _, _, lssem = bufs[(kslab - 1) % 2]
            pltpu.make_async_copy(lrows, acc.at[dslab.at[kslab - 1]],
                                  lssem).wait()
            return carry
        lax.fori_loop(0, nslabs, slab, 0)
        plsc.subcore_barrier()
        pltpu.sync_copy(acc.at[pl.ds(s * rpt, rpt)],
                        out_hbm.at[c, pl.ds(s * rpt, rpt)])

    return k(x, ea, src3d, dst3d)


def _pick_kslab(chunks, cap=25):
    for kk in range(min(chunks, cap), 0, -1):
        if chunks % kk == 0:
            return kk
    return 1


def _ceil_to(v, m):
    return -(-v // m) * m


def kernel(x, edge_attr, frag_ea, cond, params, edge_index, atom_to_frag,
           frag_ei, frag_batch):
    pa = params['atom']
    pf = params['frag']
    N, _ = x.shape
    E = edge_attr.shape[0]
    H = pa['proj_W'].shape[1]
    F = frag_batch.shape[0]
    EF = frag_ea.shape[0]
    HF = pf['proj_W'].shape[1]
    LA = pa['edge_W'].shape[0]
    LF = pf['edge_W'].shape[0]

    ch_a = 80
    ch_f = 64
    e_pad = _ceil_to(E, 32 * ch_a)
    n_acc = _ceil_to(N + 1, 16 * _ZR)
    ef_pad = _ceil_to(EF, 32 * ch_f)
    f_acc = _ceil_to(F + 1, 16 * _ZR)
    ks_a = _pick_kslab((e_pad // 32) // ch_a)
    ks_f = _pick_kslab((ef_pad // 32) // ch_f)

    srcA = jnp.concatenate(
        [edge_index[0].astype(jnp.int32),
         jnp.zeros((e_pad - E,), jnp.int32)]).reshape(-1, ks_a, ch_a)
    dstA = jnp.concatenate(
        [edge_index[1].astype(jnp.int32),
         jnp.full((e_pad - E,), N, jnp.int32)]).reshape(-1, ks_a, ch_a)
    srcF = jnp.concatenate(
        [frag_ei[0].astype(jnp.int32),
         jnp.zeros((ef_pad - EF,), jnp.int32)]).reshape(-1, ks_f, ch_f)
    dstF = jnp.concatenate(
        [frag_ei[1].astype(jnp.int32),
         jnp.full((ef_pad - EF,), F, jnp.int32)]).reshape(-1, ks_f, ch_f)

    amax_a = _absmax(edge_attr, 8000)[0, 0]
    amax_f = _absmax(frag_ea, 8000)[0, 0]

    def _scales(amax, ws, bs, L):
        s_list = []
        for l in range(L):
            bound = (amax * jnp.max(jnp.sum(jnp.abs(ws[l]), axis=0))
                     + jnp.max(jnp.abs(bs[l])))
            s_list.append(32000.0 / jnp.maximum(bound, 1e-6))
        return s_list

    s_a = _scales(amax_a, pa['edge_W'], pa['edge_b'], LA)
    s_f = _scales(amax_f, pf['edge_W'], pf['edge_b'], LF)

    h = _linear(x, pa['proj_W'], pa['proj_b'], s_a[0], 1000)
    for l in range(LA):
        ea_l = _edge_embed(edge_attr, pa['edge_W'][l], pa['edge_b'][l],
                           s_a[l], e_pad, 4000)
        agg2 = _sc_edge_pass(h, ea_l, srcA, dstA, e_pad, n_acc, ch_a, ks_a)
        s_out = s_a[l + 1] if l + 1 < LA else jnp.float32(1.0)
        scal = jnp.stack([pa['eps'][l], 1.0 / s_a[l], s_out])
        h = _gine_mlp(h, agg2, scal, pa['mlp_W1'][l], pa['mlp_b1'][l],
                      pa['mlp_W2'][l], pa['mlp_b2'][l], pa['ln_g'][l],
                      pa['ln_b'][l], 1000)

    hw = _pool(atom_to_frag.astype(jnp.int32), h,
               params['frag_proj_W'], params['frag_proj_b'],
               pf['proj_W'], pf['proj_b'], s_f[0], F, 400)

    for l in range(LF):
        ew_pad = jnp.pad(pf['edge_W'][l], ((0, 0), (0, H - HF)))
        eb_pad = jnp.pad(pf['edge_b'][l], ((0, H - HF),))
        eaf_l = _edge_embed(frag_ea, ew_pad, eb_pad, s_f[l], ef_pad, 4096)
        hwp = jnp.pad(hw, ((0, 0), (0, H - HF)))
        agg2 = _sc_edge_pass(hwp, eaf_l, srcF, dstF, ef_pad, f_acc,
                             ch_f, ks_f)
        s_out = s_f[l + 1] if l + 1 < LF else jnp.float32(1.0)
        scal = jnp.stack([pf['eps'][l], 1.0 / s_f[l], s_out])
        hw = _gine_mlp(hw, agg2, scal, pf['mlp_W1'][l],
                       pf['mlp_b1'][l], pf['mlp_W2'][l], pf['mlp_b2'][l],
                       pf['ln_g'][l], pf['ln_b'][l], 1000)

    dg, deltas = _final(hw, frag_batch.astype(jnp.int32), cond, params)
    return dg, deltas.reshape(-1)
